# baseline (speedup 1.0000x reference)
import functools
import jax
import jax.numpy as jnp
from jax import lax
from jax.experimental import pallas as pl
from jax.experimental.pallas import tpu as pltpu
from jax.experimental.pallas import tpu_sc as plsc

B0, B1, V = 32, 8, 128256
R = B0 * B1
NW = 32
ROWS_PER_W = R // NW
NCH = 8
CHUNK = V // NCH
NVREG = CHUNK // 16
UNROLL = 6
NSTEPS = ROWS_PER_W * NCH
NEG_INF = float("-inf")
BIG = 2147483647


def _scan_chunk(buf, base, m, bi, iota):
    iv0 = jnp.full((16,), base, jnp.int32) + iota

    def body(j, carry):
        m, bi, iv = carry
        off = j * (16 * UNROLL)
        for u in range(UNROLL):
            v = buf[pl.ds(off + u * 16, 16)]
            p = v > m
            m = jnp.where(p, v, m)
            bi = jnp.where(p, iv, bi)
            iv = iv + 16
        return m, bi, iv

    m, bi, _ = lax.fori_loop(0, NVREG // UNROLL, body, (m, bi, iv0))
    return m, bi


def _sortable_i32(v):
    k = plsc.bitcast(v + 0.0, jnp.int32)
    flip = jnp.where(k < 0, jnp.full((16,), 0x7FFFFFFF, jnp.int32),
                     jnp.zeros((16,), jnp.int32))
    return k ^ flip


def _sc_argmax(x_hbm, out_hbm, buf0, buf1, res_v, sem0, sem1):
    wid = lax.axis_index("s") * 2 + lax.axis_index("c")
    base = wid * (ROWS_PER_W * V)
    iota = lax.iota(jnp.int32, 16)
    bufs = (buf0, buf1)
    sems = (sem0, sem1)

    def start(step):
        return pltpu.async_copy(
            x_hbm.at[pl.ds(base + step * CHUNK, CHUNK)],
            bufs[step % 2], sems[step % 2])

    pending = start(0)
    m = jnp.full((16,), NEG_INF, jnp.float32)
    bi = jnp.zeros((16,), jnp.int32)
    res = jnp.zeros((16,), jnp.int32)
    for step in range(NSTEPS):
        r, c = divmod(step, NCH)
        nxt = start(step + 1) if step + 1 < NSTEPS else None
        pending.wait()
        pending = nxt
        m, bi = _scan_chunk(bufs[step % 2], c * CHUNK, m, bi, iota)
        if c == NCH - 1:
            kk = m

            def merge(a, b):
                ka, ia = a
                kb, ib = b
                better = (kb > ka) | ((kb == ka) & (ib < ia))
                return (lax.select(better, kb, ka), lax.select(better, ib, ia))

            pairs = [(kk[j], bi[j]) for j in range(16)]
            while len(pairs) > 1:
                pairs = [merge(pairs[i], pairs[i + 1])
                         for i in range(0, len(pairs), 2)]
            ridx = pairs[0][1]
            res = jnp.where(iota == r, jnp.full((16,), ridx, jnp.int32), res)
            m = jnp.full((16,), NEG_INF, jnp.float32)
            bi = jnp.zeros((16,), jnp.int32)

    res_v[...] = res
    pltpu.sync_copy(res_v, out_hbm.at[wid])


def kernel(logits):
    x = logits.reshape(R * V)
    mesh = plsc.VectorSubcoreMesh(core_axis_name="c", subcore_axis_name="s")
    out = pl.kernel(
        _sc_argmax,
        out_type=jax.ShapeDtypeStruct((NW, 16), jnp.int32),
        mesh=mesh,
        scratch_types=[
            pltpu.VMEM((CHUNK,), jnp.float32),
            pltpu.VMEM((CHUNK,), jnp.float32),
            pltpu.VMEM((16,), jnp.int32),
            pltpu.SemaphoreType.DMA,
            pltpu.SemaphoreType.DMA,
        ],
    )(x)
    return out[:, :ROWS_PER_W].reshape(B0, B1)
